# baseline (speedup 1.0000x reference)
import functools

import jax
import jax.numpy as jnp
from jax import lax
from jax.experimental import pallas as pl
from jax.experimental.pallas import tpu as pltpu
from jax.experimental.pallas import tpu_sc as plsc

EMBED_DIM = 32
NUM_WORKERS = 32
CHUNK_B = 16


def _sc_embedding_gather(x, table, S8):
    B = x.shape[0]
    b_per_w = B // NUM_WORKERS
    n_chunks = b_per_w // CHUNK_B

    mesh = plsc.VectorSubcoreMesh(core_axis_name="c", subcore_axis_name="s")

    Sp = x.shape[1]

    @functools.partial(
        pl.kernel,
        out_type=jax.ShapeDtypeStruct((B, S8, EMBED_DIM), jnp.float32),
        mesh=mesh,
        scratch_types=[
            pltpu.VMEM((b_per_w, Sp), jnp.int32),
            pltpu.VMEM((2, CHUNK_B, S8, EMBED_DIM), jnp.float32),
            pltpu.SemaphoreType.DMA,
            pltpu.SemaphoreType.DMA,
        ],
        compiler_params=pltpu.CompilerParams(use_tc_tiling_on_sc=False),
    )
    def body(x_hbm, tab_hbm, out_hbm, idx_v, rows_v, gsem, ssem):
        wid = lax.axis_index("s") * 2 + lax.axis_index("c")
        base = wid * b_per_w
        pltpu.sync_copy(x_hbm.at[pl.ds(base, b_per_w)], idx_v)

        def fire_gather(c, slot):
            for r in range(CHUNK_B):
                pltpu.make_async_copy(
                    tab_hbm.at[idx_v.at[c * CHUNK_B + r, pl.ds(0, S8)]],
                    rows_v.at[slot, r],
                    gsem,
                ).start()

        def wait_gather(c, slot):
            for r in range(CHUNK_B):
                pltpu.make_async_copy(
                    tab_hbm.at[idx_v.at[c * CHUNK_B + r, pl.ds(0, S8)]],
                    rows_v.at[slot, r],
                    gsem,
                ).wait()

        def store_desc(c, slot):
            return pltpu.make_async_copy(
                rows_v.at[slot],
                out_hbm.at[pl.ds(base + c * CHUNK_B, CHUNK_B)],
                ssem,
            )

        fire_gather(0, 0)

        def chunk_body(c, carry):
            slot = lax.rem(c, 2)
            wait_gather(c, slot)

            @pl.when(c >= 1)
            def _():
                store_desc(c - 1, 1 - slot).wait()

            @pl.when(c + 1 < n_chunks)
            def _():
                fire_gather(c + 1, 1 - slot)

            store_desc(c, slot).start()
            return carry

        lax.fori_loop(0, n_chunks, chunk_body, 0)
        store_desc(n_chunks - 1, lax.rem(n_chunks - 1, 2)).wait()

    return body(x, table)


def kernel(x, raw_weight):
    B, S = x.shape
    S8 = (S + 7) // 8 * 8
    xp = jnp.pad(x.astype(jnp.int32), ((0, 0), (0, 128 - S)))
    out = _sc_embedding_gather(xp, raw_weight, S8)
    return out[:, :S, :]


# speedup vs baseline: 2.0223x; 2.0223x over previous
import functools

import jax
import jax.numpy as jnp
from jax import lax
from jax.experimental import pallas as pl
from jax.experimental.pallas import tpu as pltpu
from jax.experimental.pallas import tpu_sc as plsc

EMBED_DIM = 32
NUM_WORKERS = 32
CHUNK_B = 8
LANES = 16


def _sc_embedding_gather(x, table, S):
    B, Sp = x.shape
    b_per_w = B // NUM_WORKERS
    n_chunks = b_per_w // CHUNK_B
    n_vecs = b_per_w * S // LANES

    mesh = plsc.VectorSubcoreMesh(core_axis_name="c", subcore_axis_name="s")

    @functools.partial(
        pl.kernel,
        out_type=jax.ShapeDtypeStruct((B, S, EMBED_DIM), jnp.float32),
        mesh=mesh,
        scratch_types=[
            pltpu.VMEM((b_per_w, Sp), jnp.int32),
            pltpu.VMEM((b_per_w, S), jnp.int32),
            pltpu.VMEM((2, CHUNK_B, S, EMBED_DIM), jnp.float32),
            pltpu.SemaphoreType.DMA,
            pltpu.SemaphoreType.DMA,
        ],
        compiler_params=pltpu.CompilerParams(
            use_tc_tiling_on_sc=False, needs_layout_passes=False
        ),
    )
    def body(x_hbm, tab_hbm, out_hbm, raw_v, idx_v, rows_v, gsem, ssem):
        wid = lax.axis_index("s") * 2 + lax.axis_index("c")
        base = wid * b_per_w
        pltpu.sync_copy(x_hbm.at[pl.ds(base, b_per_w)], raw_v)

        lane = lax.iota(jnp.int32, LANES)

        def compact_body(r, carry):
            row = jnp.full((LANES,), r, jnp.int32)
            for k in range((S + LANES - 1) // LANES):
                col = k * LANES + lane
                v = raw_v[r, pl.ds(k * LANES, LANES)]
                plsc.store_scatter(idx_v, [row, col], v, mask=col < S)
            return carry

        lax.fori_loop(0, b_per_w, compact_body, 0)

        def fire_gather(c, slot):
            for r in range(CHUNK_B):
                pltpu.make_async_copy(
                    tab_hbm.at[idx_v.at[c * CHUNK_B + r]],
                    rows_v.at[slot, r],
                    gsem,
                ).start()

        def wait_gather(c, slot):
            for r in range(CHUNK_B):
                pltpu.make_async_copy(
                    tab_hbm.at[idx_v.at[c * CHUNK_B + r]],
                    rows_v.at[slot, r],
                    gsem,
                ).wait()

        def store_desc(c, slot):
            return pltpu.make_async_copy(
                rows_v.at[slot],
                out_hbm.at[pl.ds(base + c * CHUNK_B, CHUNK_B)],
                ssem,
            )

        fire_gather(0, 0)

        def chunk_body(c, carry):
            slot = lax.rem(c, 2)
            wait_gather(c, slot)

            @pl.when(c >= 1)
            def _():
                store_desc(c - 1, 1 - slot).wait()

            @pl.when(c + 1 < n_chunks)
            def _():
                fire_gather(c + 1, 1 - slot)

            store_desc(c, slot).start()
            return carry

        lax.fori_loop(0, n_chunks, chunk_body, 0)
        store_desc(n_chunks - 1, lax.rem(n_chunks - 1, 2)).wait()

    return body(x, table)


def kernel(x, raw_weight):
    B, S = x.shape
    xp = jnp.pad(x.astype(jnp.int32), ((0, 0), (0, 128 - S)))
    return _sc_embedding_gather(xp, raw_weight, S)
